# baseline (speedup 1.0000x reference)
import functools

import jax
import jax.numpy as jnp
from jax import lax
from jax.experimental import pallas as pl
from jax.experimental.pallas import tpu as pltpu
from jax.experimental.pallas import tpu_sc as plsc

_DIM = 50
_DIMB = 64
_DIMW = 32
_NPK = _DIM // 2
_BATCH = 16384
_NC = 2
_NS = 16
_L = 16
_NW = _NC * _NS
_BPW = _BATCH // _NW
_CH = 128
_NCH = _BPW // _CH

_mesh = plsc.VectorSubcoreMesh(core_axis_name="c", subcore_axis_name="s")


@functools.partial(
    pl.kernel,
    out_type=jax.ShapeDtypeStruct((_BATCH,), jnp.float32),
    mesh=_mesh,
    scratch_types=[
        pltpu.VMEM((_BPW,), jnp.int32),
        pltpu.VMEM((_BPW,), jnp.int32),
        pltpu.VMEM((_BPW, _DIMW), jnp.int32),
        pltpu.VMEM((_BPW, _DIMW), jnp.int32),
        pltpu.VMEM((_BPW,), jnp.float32),
        pltpu.SemaphoreType.DMA,
    ],
    compiler_params=pltpu.CompilerParams(
        use_tc_tiling_on_sc=False, needs_layout_passes=False),
)
def _sc_dot(iw_hbm, ic_hbm, words_hbm, ctx_hbm, out_hbm,
            iw_v, ic_v, wrows, crows, outv, sem):
    wid = lax.axis_index("s") * _NC + lax.axis_index("c")
    base = wid * _BPW

    pltpu.sync_copy(iw_hbm.at[pl.ds(base, _BPW)], iw_v)
    pltpu.sync_copy(ic_hbm.at[pl.ds(base, _BPW)], ic_v)

    copies = []
    for j in range(_NCH):
        s = pl.ds(j * _CH, _CH)
        copies.append(pltpu.async_copy(words_hbm.at[iw_v.at[s]], wrows.at[s], sem))
        copies.append(pltpu.async_copy(ctx_hbm.at[ic_v.at[s]], crows.at[s], sem))
    for cp in copies:
        cp.wait()

    himask = jnp.full((_L,), -65536, jnp.int32)

    def group(g, carry):
        rows = g * _L + lax.iota(jnp.int32, _L)
        acc = jnp.zeros((_L,), jnp.float32)
        for k in range(_NPK):
            col = jnp.full((_L,), k, jnp.int32)
            w = plsc.load_gather(wrows, [rows, col])
            c = plsc.load_gather(crows, [rows, col])
            wlo = plsc.bitcast(lax.shift_left(w, 16), jnp.float32)
            clo = plsc.bitcast(lax.shift_left(c, 16), jnp.float32)
            whi = plsc.bitcast(w & himask, jnp.float32)
            chi = plsc.bitcast(c & himask, jnp.float32)
            acc = acc + wlo * clo + whi * chi
        outv[pl.ds(g * _L, _L)] = acc
        return carry

    lax.fori_loop(0, _BPW // _L, group, 0)

    pltpu.sync_copy(outv, out_hbm.at[pl.ds(base, _BPW)])


def _pack(t):
    tb = jnp.pad(t.astype(jnp.bfloat16), ((0, 0), (0, _DIMB - _DIM)))
    return lax.bitcast_convert_type(
        tb.reshape(t.shape[0], _DIMW, 2), jnp.int32)


def kernel(i_w, i_c, words, contexts):
    out = _sc_dot(i_w.astype(jnp.int32), i_c.astype(jnp.int32),
                  _pack(words), _pack(contexts))
    return out.reshape(_BATCH, 1, 1)


# speedup vs baseline: 1.2431x; 1.2431x over previous
import functools

import jax
import jax.numpy as jnp
from jax import lax
from jax.experimental import pallas as pl
from jax.experimental.pallas import tpu as pltpu
from jax.experimental.pallas import tpu_sc as plsc

_DIM = 50
_DIMP = 56
_BATCH = 16384
_NC = 2
_NS = 16
_L = 16
_NW = _NC * _NS
_BPW = _BATCH // _NW
_CH = 128
_NCH = _BPW // _CH

_mesh = plsc.VectorSubcoreMesh(core_axis_name="c", subcore_axis_name="s")


@functools.partial(
    pl.kernel,
    out_type=jax.ShapeDtypeStruct((_BATCH,), jnp.float32),
    mesh=_mesh,
    scratch_types=[
        pltpu.VMEM((_BPW,), jnp.int32),
        pltpu.VMEM((_BPW,), jnp.int32),
        pltpu.VMEM((_BPW, _DIMP), jnp.float32),
        pltpu.VMEM((_BPW, _DIMP), jnp.float32),
        pltpu.VMEM((_BPW,), jnp.float32),
        pltpu.SemaphoreType.DMA,
        pltpu.SemaphoreType.DMA,
    ],
    compiler_params=pltpu.CompilerParams(
        use_tc_tiling_on_sc=False, needs_layout_passes=False),
)
def _sc_dot(iw_hbm, ic_hbm, words_hbm, ctx_hbm, out_hbm,
            iw_v, ic_v, wrows, crows, outv, sem_a, sem_b):
    wid = lax.axis_index("s") * _NC + lax.axis_index("c")
    base = wid * _BPW

    pltpu.sync_copy(iw_hbm.at[pl.ds(base, _BPW)], iw_v)
    pltpu.sync_copy(ic_hbm.at[pl.ds(base, _BPW)], ic_v)

    sems = [sem_a, sem_b]

    def issue(j):
        s = pl.ds(j * _CH, _CH)
        return (
            pltpu.async_copy(words_hbm.at[iw_v.at[s]], wrows.at[s], sems[j % 2]),
            pltpu.async_copy(ctx_hbm.at[ic_v.at[s]], crows.at[s], sems[j % 2]),
        )

    pending = issue(0)
    for j in range(_NCH):
        nxt = issue(j + 1) if j + 1 < _NCH else None
        pending[0].wait()
        pending[1].wait()
        pending = nxt

        def group(g, carry, j=j):
            rows = j * _CH + g * _L + lax.iota(jnp.int32, _L)
            acc = jnp.zeros((_L,), jnp.float32)
            for d in range(_DIM):
                col = jnp.full((_L,), d, jnp.int32)
                w = plsc.load_gather(wrows, [rows, col])
                c = plsc.load_gather(crows, [rows, col])
                acc = acc + w * c
            outv[pl.ds(j * _CH + g * _L, _L)] = acc
            return carry

        lax.fori_loop(0, _CH // _L, group, 0)

    pltpu.sync_copy(outv, out_hbm.at[pl.ds(base, _BPW)])


def kernel(i_w, i_c, words, contexts):
    wp = jnp.pad(words, ((0, 0), (0, _DIMP - _DIM)))
    cp = jnp.pad(contexts, ((0, 0), (0, _DIMP - _DIM)))
    out = _sc_dot(i_w.astype(jnp.int32), i_c.astype(jnp.int32), wp, cp)
    return out.reshape(_BATCH, 1, 1)


# speedup vs baseline: 1.2606x; 1.0141x over previous
import functools

import jax
import jax.numpy as jnp
from jax import lax
from jax.experimental import pallas as pl
from jax.experimental.pallas import tpu as pltpu
from jax.experimental.pallas import tpu_sc as plsc

_DIM = 50
_DIMP = 56
_BATCH = 16384
_NC = 2
_NS = 16
_L = 16
_NW = _NC * _NS
_BPW = _BATCH // _NW
_CH = 128
_NCH = _BPW // _CH

_mesh = plsc.VectorSubcoreMesh(core_axis_name="c", subcore_axis_name="s")


@functools.partial(
    pl.kernel,
    out_type=jax.ShapeDtypeStruct((_BATCH,), jnp.float32),
    mesh=_mesh,
    scratch_types=[
        pltpu.VMEM((_BPW,), jnp.int32),
        pltpu.VMEM((_BPW,), jnp.int32),
        pltpu.VMEM((_BPW, _DIMP), jnp.float32),
        pltpu.VMEM((_BPW, _DIMP), jnp.float32),
        pltpu.VMEM((_BPW,), jnp.float32),
        pltpu.SemaphoreType.DMA,
    ],
    compiler_params=pltpu.CompilerParams(
        use_tc_tiling_on_sc=False, needs_layout_passes=False),
)
def _sc_dot(iw_hbm, ic_hbm, words_hbm, ctx_hbm, out_hbm,
            iw_v, ic_v, wrows, crows, outv, sem):
    wid = lax.axis_index("s") * _NC + lax.axis_index("c")
    base = wid * _BPW

    pltpu.sync_copy(iw_hbm.at[pl.ds(base, _BPW)], iw_v)
    pltpu.sync_copy(ic_hbm.at[pl.ds(base, _BPW)], ic_v)

    copies = []
    for j in range(_NCH):
        s = pl.ds(j * _CH, _CH)
        copies.append(pltpu.async_copy(words_hbm.at[iw_v.at[s]], wrows.at[s], sem))
        copies.append(pltpu.async_copy(ctx_hbm.at[ic_v.at[s]], crows.at[s], sem))
    for cp in copies:
        cp.wait()

    def group(g, carry):
        rows = g * _L + lax.iota(jnp.int32, _L)
        acc = jnp.zeros((_L,), jnp.float32)
        for d in range(_DIM):
            col = jnp.full((_L,), d, jnp.int32)
            w = plsc.load_gather(wrows, [rows, col])
            c = plsc.load_gather(crows, [rows, col])
            acc = acc + w * c
        outv[pl.ds(g * _L, _L)] = acc
        return carry

    lax.fori_loop(0, _BPW // _L, group, 0)

    pltpu.sync_copy(outv, out_hbm.at[pl.ds(base, _BPW)])


def kernel(i_w, i_c, words, contexts):
    wp = jnp.pad(words, ((0, 0), (0, _DIMP - _DIM)))
    cp = jnp.pad(contexts, ((0, 0), (0, _DIMP - _DIM)))
    out = _sc_dot(i_w.astype(jnp.int32), i_c.astype(jnp.int32), wp, cp)
    return out.reshape(_BATCH, 1, 1)
